# Initial kernel scaffold; baseline (speedup 1.0000x reference)
import jax
import jax.numpy as jnp
from jax.experimental import pallas as pl


def kernel(x, edge_index, Wq, Wk, Wv, Wo, Wskip, Wg, g1, b1, g2, b2, W1, W2):
    raise NotImplementedError("write your pallas kernel here")


# speedup vs baseline: 44.6832x; 44.6832x over previous
import functools

import jax
import jax.numpy as jnp
from jax import lax
from jax.experimental import pallas as pl
from jax.experimental.pallas import tpu as pltpu
from jax.experimental.pallas import tpu_sc as plsc

N = 10000
E = 320000
D = 128
H = 8
HD = 16
QSCALE = float(HD ** 0.5)

NC = 2
NS = 16
NW = NC * NS
EPW = E // NW
CH = 80
NCHUNK = EPW // CH
GPC = CH // 16
NP = 10240
RPT = NP // NS
DR = NP // 8
DPT = DR // NS
ZB = 40
NZB = RPT // ZB
DZB = DPT // ZB


def _proj_body(x_ref, wq_ref, wk_ref, wv_ref, ws_ref, q_ref, k_ref, v_ref,
               s_ref):
    xb = x_ref[...]
    q_ref[...] = QSCALE * jnp.dot(
        xb, wq_ref[...], preferred_element_type=jnp.float32)
    k_ref[...] = jnp.dot(xb, wk_ref[...], preferred_element_type=jnp.float32)
    v_ref[...] = jnp.dot(xb, wv_ref[...], preferred_element_type=jnp.float32)
    s_ref[...] = jnp.dot(xb, ws_ref[...], preferred_element_type=jnp.float32)


def _proj(x, Wq, Wk, Wv, Wskip, bm=2000):
    grid = (N // bm,)
    blk = pl.BlockSpec((bm, D), lambda i: (i, 0))
    wblk = pl.BlockSpec((D, D), lambda i: (0, 0))
    out = jax.ShapeDtypeStruct((N, D), jnp.float32)
    return pl.pallas_call(
        _proj_body,
        grid=grid,
        in_specs=[blk, wblk, wblk, wblk, wblk],
        out_specs=[blk, blk, blk, blk],
        out_shape=[out, out, out, out],
    )(x, Wq, Wk, Wv, Wskip)


_GDN = lax.GatherDimensionNumbers(
    offset_dims=(), collapsed_slice_dims=(0,), start_index_map=(0,))


def _shuffle(x, idx):
    return lax.gather(x, idx.reshape(16, 1), _GDN, (1,),
                      mode=lax.GatherScatterMode.PROMISE_IN_BOUNDS)


def _bcast_lane(x, j):
    return _shuffle(x, jnp.full((16,), j, dtype=jnp.int32))


def _edge_body(q_hbm, k_hbm, v_hbm, src_hbm, dst_hbm, wv_out, den_out,
               acc_wv, acc_den, src_v, dst_v, ddiv_v, band_v, qr, kr, vr,
               bounce, sem):
    c = lax.axis_index("c")
    s = lax.axis_index("s")
    wid = s * NC + c
    lane = lax.iota(jnp.int32, 16)
    zvec = jnp.zeros((16,), jnp.float32)

    def zrow(r, _):
        for cb in range(D // 16):
            bounce[r, pl.ds(16 * cb, 16)] = zvec
        return 0

    lax.fori_loop(0, ZB, zrow, 0)
    r0 = s * RPT
    for blk in range(NZB):
        pltpu.sync_copy(bounce, acc_wv.at[pl.ds(r0 + blk * ZB, ZB)])
    for blk in range(DZB):
        pltpu.sync_copy(bounce, acc_den.at[pl.ds(s * DPT + blk * ZB, ZB)])
    plsc.subcore_barrier()

    e_base = wid * EPW

    def chunk(i, _):
        off = pl.multiple_of(e_base + i * CH, 8)
        pltpu.sync_copy(src_hbm.at[pl.ds(off, CH)], src_v)
        pltpu.sync_copy(dst_hbm.at[pl.ds(off, CH)], dst_v)
        cp_q = pltpu.async_copy(q_hbm.at[src_v], qr, sem)
        cp_k = pltpu.async_copy(k_hbm.at[dst_v], kr, sem)
        cp_v = pltpu.async_copy(v_hbm.at[src_v], vr, sem)
        cp_q.wait()
        cp_k.wait()
        cp_v.wait()

        def group(g, _):
            dstg = dst_v[pl.ds(g * 16, 16)]
            ddiv_v[pl.ds(g * 16, 16)] = lax.shift_right_logical(dstg, 3)
            for j in range(16):
                e = g * 16 + j
                band_v[pl.ds(0, 16)] = _bcast_lane(dstg, j) & 7
                band = band_v[pl.ds(0, 16)]
                den_acc = zvec
                for h in range(H):
                    qv = qr[e, pl.ds(16 * h, 16)]
                    kv = kr[e, pl.ds(16 * h, 16)]
                    a = qv * kv
                    for sh in (8, 4, 2, 1):
                        a = a + _shuffle(a, lane ^ sh)
                    a = jnp.clip(a, -20.0, 20.0)
                    exv = jnp.exp(a)
                    vr[e, pl.ds(16 * h, 16)] = exv * vr[e, pl.ds(16 * h, 16)]
                    den_acc = jnp.where(lane == h, exv, den_acc)
                for cb in range(8):
                    qr[e, pl.ds(16 * cb, 16)] = jnp.where(
                        band == cb, den_acc, zvec)
            return 0

        lax.fori_loop(0, GPC, group, 0)
        pltpu.sync_copy(vr, acc_wv.at[dst_v], add=True)
        pltpu.sync_copy(qr, acc_den.at[ddiv_v], add=True)
        return 0

    lax.fori_loop(0, NCHUNK, chunk, 0)
    plsc.subcore_barrier()

    for blk in range(NZB):
        rr = r0 + blk * ZB
        pltpu.sync_copy(acc_wv.at[pl.ds(rr, ZB)], bounce)
        pltpu.sync_copy(bounce, wv_out.at[c, pl.ds(rr, ZB)])
    for blk in range(DZB):
        dd = s * DPT + blk * ZB
        pltpu.sync_copy(acc_den.at[pl.ds(dd, ZB)], bounce)
        pltpu.sync_copy(bounce, den_out.at[c, pl.ds(dd, ZB)])


_edge = pl.kernel(
    _edge_body,
    out_type=[
        jax.ShapeDtypeStruct((NC, NP, D), jnp.float32),
        jax.ShapeDtypeStruct((NC, DR, D), jnp.float32),
    ],
    mesh=plsc.VectorSubcoreMesh(core_axis_name="c", subcore_axis_name="s"),
    scratch_types=[
        pltpu.VMEM_SHARED((NP, D), jnp.float32),
        pltpu.VMEM_SHARED((DR, D), jnp.float32),
        pltpu.VMEM((CH,), jnp.int32),
        pltpu.VMEM((CH,), jnp.int32),
        pltpu.VMEM((CH,), jnp.int32),
        pltpu.VMEM((16,), jnp.int32),
        pltpu.VMEM((CH, D), jnp.float32),
        pltpu.VMEM((CH, D), jnp.float32),
        pltpu.VMEM((CH, D), jnp.float32),
        pltpu.VMEM((ZB, D), jnp.float32),
        pltpu.SemaphoreType.DMA,
    ],
)


def _epi_body(wv0_ref, wv1_ref, d0_ref, d1_ref, skip_ref, wo_ref, wg_ref,
              g1_ref, b1_ref, g2_ref, b2_ref, w1_ref, w2_ref, out_ref):
    den16 = d0_ref[...] + d1_ref[...]
    jrow = lax.broadcasted_iota(jnp.int32, (16, D), 0)
    lcol = lax.broadcasted_iota(jnp.int32, (16, D), 1) // HD
    emat = (jrow == lcol).astype(jnp.float32)
    dexp = jnp.dot(den16, emat, preferred_element_type=jnp.float32)
    dexp = jnp.where(dexp > 0.0, dexp, 1.0)
    agg = (wv0_ref[...] + wv1_ref[...]) / dexp

    rst = jnp.dot(agg, wo_ref[...], preferred_element_type=jnp.float32)
    skip = skip_ref[...]
    wg = wg_ref[...]
    ga = wg[0] + wg[2]
    gb = wg[1] - wg[2]
    glog = (jnp.sum(rst * ga, axis=-1, keepdims=True)
            + jnp.sum(skip * gb, axis=-1, keepdims=True))
    gate = 1.0 / (1.0 + jnp.exp(-glog))
    hmid = rst * gate + skip * (1.0 - gate)

    mu = jnp.mean(hmid, axis=-1, keepdims=True)
    var = jnp.mean((hmid - mu) ** 2, axis=-1, keepdims=True)
    hn = (hmid - mu) * lax.rsqrt(var + 1e-5) * g1_ref[...] + b1_ref[...]

    mu2 = jnp.mean(hn, axis=-1, keepdims=True)
    var2 = jnp.mean((hn - mu2) ** 2, axis=-1, keepdims=True)
    h2 = (hn - mu2) * lax.rsqrt(var2 + 1e-5) * g2_ref[...] + b2_ref[...]

    ffn = jnp.dot(
        jnp.maximum(
            jnp.dot(h2, w1_ref[...], preferred_element_type=jnp.float32), 0.0),
        w2_ref[...], preferred_element_type=jnp.float32)
    out_ref[...] = hn + ffn


def _epilogue(wv, den2, skip, Wo, Wg3, g1, b1, g2, b2, W1, W2, bm=2000):
    grid = (N // bm,)
    blk = pl.BlockSpec((bm, D), lambda i: (i, 0))
    dblk = pl.BlockSpec((bm, 16), lambda i: (i, 0))
    wblk = pl.BlockSpec((D, D), lambda i: (0, 0))
    gblk = pl.BlockSpec((3, D), lambda i: (0, 0))
    vblk = pl.BlockSpec((1, D), lambda i: (0, 0))
    return pl.pallas_call(
        _epi_body,
        grid=grid,
        in_specs=[blk, blk, dblk, dblk, blk, wblk, gblk,
                  vblk, vblk, vblk, vblk, wblk, wblk],
        out_specs=blk,
        out_shape=jax.ShapeDtypeStruct((N, D), jnp.float32),
    )(wv[0], wv[1], den2[0], den2[1], skip, Wo, Wg3, g1, b1, g2, b2, W1, W2)


def kernel(x, edge_index, Wq, Wk, Wv, Wo, Wskip, Wg, g1, b1, g2, b2, W1, W2):
    src = edge_index[0].astype(jnp.int32)
    dst = edge_index[1].astype(jnp.int32)
    q, k, v, skip = _proj(x, Wq, Wk, Wv, Wskip)
    wv, den = _edge(q, k, v, src, dst)
    den2 = den.reshape(NC, NP, 16)
    Wg3 = Wg[:, 0].reshape(3, D)
    return _epilogue(wv, den2, skip, Wo, Wg3,
                     g1.reshape(1, D), b1.reshape(1, D),
                     g2.reshape(1, D), b2.reshape(1, D), W1, W2)


# speedup vs baseline: 54.1893x; 1.2127x over previous
import functools

import jax
import jax.numpy as jnp
from jax import lax
from jax.experimental import pallas as pl
from jax.experimental.pallas import tpu as pltpu
from jax.experimental.pallas import tpu_sc as plsc

N = 10000
E = 320000
D = 128
H = 8
HD = 16
QSCALE = float(HD ** 0.5)

NC = 2
NS = 16
NW = NC * NS
EPW = E // NW
CH = 80
NCHUNK = EPW // CH
GPC = CH // 16
NP = 10240
RPT = NP // NS
DR = NP // 8
DPT = DR // NS
ZB = 40
NZB = RPT // ZB
DZB = DPT // ZB


def _proj_body(x_ref, wq_ref, wk_ref, wv_ref, ws_ref, q_ref, k_ref, v_ref,
               s_ref):
    xb = x_ref[...]
    q_ref[...] = QSCALE * jnp.dot(
        xb, wq_ref[...], preferred_element_type=jnp.float32)
    k_ref[...] = jnp.dot(xb, wk_ref[...], preferred_element_type=jnp.float32)
    v_ref[...] = jnp.dot(xb, wv_ref[...], preferred_element_type=jnp.float32)
    s_ref[...] = jnp.dot(xb, ws_ref[...], preferred_element_type=jnp.float32)


def _proj(x, Wq, Wk, Wv, Wskip, bm=2000):
    grid = (N // bm,)
    blk = pl.BlockSpec((bm, D), lambda i: (i, 0))
    wblk = pl.BlockSpec((D, D), lambda i: (0, 0))
    out = jax.ShapeDtypeStruct((N, D), jnp.float32)
    return pl.pallas_call(
        _proj_body,
        grid=grid,
        in_specs=[blk, wblk, wblk, wblk, wblk],
        out_specs=[blk, blk, blk, blk],
        out_shape=[out, out, out, out],
    )(x, Wq, Wk, Wv, Wskip)


_GDN = lax.GatherDimensionNumbers(
    offset_dims=(), collapsed_slice_dims=(0,), start_index_map=(0,))


def _shuffle(x, idx):
    return lax.gather(x, idx.reshape(16, 1), _GDN, (1,),
                      mode=lax.GatherScatterMode.PROMISE_IN_BOUNDS)


def _bcast_lane(x, j):
    return _shuffle(x, jnp.full((16,), j, dtype=jnp.int32))


POS = (0, 8, 4, 12, 2, 10, 6, 14)


def _edge_body(q_hbm, k_hbm, v_hbm, src_hbm, dst_hbm, wv_out, den_out,
               acc_wv, acc_den, src_v, dst_v, ddiv_v, band_v, qr, kr, vr,
               bounce, sem):
    c = lax.axis_index("c")
    s = lax.axis_index("s")
    wid = s * NC + c
    lane = lax.iota(jnp.int32, 16)
    zvec = jnp.zeros((16,), jnp.float32)
    onev = jnp.ones((16,), jnp.float32)
    idxden = jnp.where(
        lane < 8,
        ((lane & 1) << 3) | ((lane & 2) << 1) | ((lane & 4) >> 1),
        jnp.zeros((16,), jnp.int32))

    def zrow(r, _):
        for cb in range(D // 16):
            bounce[r, pl.ds(16 * cb, 16)] = zvec
        return 0

    lax.fori_loop(0, ZB, zrow, 0)
    r0 = s * RPT
    for blk in range(NZB):
        pltpu.sync_copy(bounce, acc_wv.at[pl.ds(r0 + blk * ZB, ZB)])
    for blk in range(DZB):
        pltpu.sync_copy(bounce, acc_den.at[pl.ds(s * DPT + blk * ZB, ZB)])
    plsc.subcore_barrier()

    e_base = wid * EPW

    def chunk(i, _):
        off = pl.multiple_of(e_base + i * CH, 8)
        pltpu.sync_copy(src_hbm.at[pl.ds(off, CH)], src_v)
        pltpu.sync_copy(dst_hbm.at[pl.ds(off, CH)], dst_v)
        cp_q = pltpu.async_copy(q_hbm.at[src_v], qr, sem)
        cp_k = pltpu.async_copy(k_hbm.at[dst_v], kr, sem)
        cp_v = pltpu.async_copy(v_hbm.at[src_v], vr, sem)
        cp_q.wait()
        cp_k.wait()
        cp_v.wait()

        def group(g, _):
            dstg = dst_v[pl.ds(g * 16, 16)]
            ddiv_v[pl.ds(g * 16, 16)] = lax.shift_right_logical(dstg, 3)
            band_v[pl.ds(0, 16)] = dstg & 7
            bandall = band_v[pl.ds(0, 16)]
            eqf = [jnp.where(bandall == cb, onev, zvec) for cb in range(8)]
            m8 = lane < 8
            m4 = (lane & 7) < 4
            m2 = (lane & 3) < 2
            for j in range(16):
                e = g * 16 + j
                aa = [qr[e, pl.ds(16 * h, 16)] * kr[e, pl.ds(16 * h, 16)]
                      for h in range(H)]
                aa = [a + _shuffle(a, lane ^ 8) for a in aa]
                bb = [jnp.where(m8, aa[2 * i], _shuffle(aa[2 * i + 1],
                                                        lane ^ 8))
                      for i in range(4)]
                bb = [b + _shuffle(b, lane ^ 4) for b in bb]
                cc = [jnp.where(m4, bb[2 * i], _shuffle(bb[2 * i + 1],
                                                        lane ^ 4))
                      for i in range(2)]
                cc = [c2 + _shuffle(c2, lane ^ 2) for c2 in cc]
                f = jnp.where(m2, cc[0], _shuffle(cc[1], lane ^ 2))
                f = f + _shuffle(f, lane ^ 1)
                exm = jnp.exp(jnp.clip(f, -20.0, 20.0))
                for h in range(H):
                    bch = _bcast_lane(exm, POS[h])
                    vr[e, pl.ds(16 * h, 16)] = bch * vr[e, pl.ds(16 * h, 16)]
                den_acc = jnp.where(m8, _shuffle(exm, idxden), zvec)
                for cb in range(8):
                    mb = _bcast_lane(eqf[cb], j)
                    qr[e, pl.ds(16 * cb, 16)] = mb * den_acc
            return 0

        lax.fori_loop(0, GPC, group, 0)
        pltpu.sync_copy(vr, acc_wv.at[dst_v], add=True)
        pltpu.sync_copy(qr, acc_den.at[ddiv_v], add=True)
        return 0

    lax.fori_loop(0, NCHUNK, chunk, 0)
    plsc.subcore_barrier()

    for blk in range(NZB):
        rr = r0 + blk * ZB
        pltpu.sync_copy(acc_wv.at[pl.ds(rr, ZB)], bounce)
        pltpu.sync_copy(bounce, wv_out.at[c, pl.ds(rr, ZB)])
    for blk in range(DZB):
        dd = s * DPT + blk * ZB
        pltpu.sync_copy(acc_den.at[pl.ds(dd, ZB)], bounce)
        pltpu.sync_copy(bounce, den_out.at[c, pl.ds(dd, ZB)])


_edge = pl.kernel(
    _edge_body,
    out_type=[
        jax.ShapeDtypeStruct((NC, NP, D), jnp.float32),
        jax.ShapeDtypeStruct((NC, DR, D), jnp.float32),
    ],
    mesh=plsc.VectorSubcoreMesh(core_axis_name="c", subcore_axis_name="s"),
    scratch_types=[
        pltpu.VMEM_SHARED((NP, D), jnp.float32),
        pltpu.VMEM_SHARED((DR, D), jnp.float32),
        pltpu.VMEM((CH,), jnp.int32),
        pltpu.VMEM((CH,), jnp.int32),
        pltpu.VMEM((CH,), jnp.int32),
        pltpu.VMEM((16,), jnp.int32),
        pltpu.VMEM((CH, D), jnp.float32),
        pltpu.VMEM((CH, D), jnp.float32),
        pltpu.VMEM((CH, D), jnp.float32),
        pltpu.VMEM((ZB, D), jnp.float32),
        pltpu.SemaphoreType.DMA,
    ],
)


def _epi_body(wv0_ref, wv1_ref, d0_ref, d1_ref, skip_ref, wo_ref, wg_ref,
              g1_ref, b1_ref, g2_ref, b2_ref, w1_ref, w2_ref, out_ref):
    den16 = d0_ref[...] + d1_ref[...]
    jrow = lax.broadcasted_iota(jnp.int32, (16, D), 0)
    lcol = lax.broadcasted_iota(jnp.int32, (16, D), 1) // HD
    emat = (jrow == lcol).astype(jnp.float32)
    dexp = jnp.dot(den16, emat, preferred_element_type=jnp.float32)
    dexp = jnp.where(dexp > 0.0, dexp, 1.0)
    agg = (wv0_ref[...] + wv1_ref[...]) / dexp

    rst = jnp.dot(agg, wo_ref[...], preferred_element_type=jnp.float32)
    skip = skip_ref[...]
    wg = wg_ref[...]
    ga = wg[0] + wg[2]
    gb = wg[1] - wg[2]
    glog = (jnp.sum(rst * ga, axis=-1, keepdims=True)
            + jnp.sum(skip * gb, axis=-1, keepdims=True))
    gate = 1.0 / (1.0 + jnp.exp(-glog))
    hmid = rst * gate + skip * (1.0 - gate)

    mu = jnp.mean(hmid, axis=-1, keepdims=True)
    var = jnp.mean((hmid - mu) ** 2, axis=-1, keepdims=True)
    hn = (hmid - mu) * lax.rsqrt(var + 1e-5) * g1_ref[...] + b1_ref[...]

    mu2 = jnp.mean(hn, axis=-1, keepdims=True)
    var2 = jnp.mean((hn - mu2) ** 2, axis=-1, keepdims=True)
    h2 = (hn - mu2) * lax.rsqrt(var2 + 1e-5) * g2_ref[...] + b2_ref[...]

    ffn = jnp.dot(
        jnp.maximum(
            jnp.dot(h2, w1_ref[...], preferred_element_type=jnp.float32), 0.0),
        w2_ref[...], preferred_element_type=jnp.float32)
    out_ref[...] = hn + ffn


def _epilogue(wv, den2, skip, Wo, Wg3, g1, b1, g2, b2, W1, W2, bm=2000):
    grid = (N // bm,)
    blk = pl.BlockSpec((bm, D), lambda i: (i, 0))
    dblk = pl.BlockSpec((bm, 16), lambda i: (i, 0))
    wblk = pl.BlockSpec((D, D), lambda i: (0, 0))
    gblk = pl.BlockSpec((3, D), lambda i: (0, 0))
    vblk = pl.BlockSpec((1, D), lambda i: (0, 0))
    return pl.pallas_call(
        _epi_body,
        grid=grid,
        in_specs=[blk, blk, dblk, dblk, blk, wblk, gblk,
                  vblk, vblk, vblk, vblk, wblk, wblk],
        out_specs=blk,
        out_shape=jax.ShapeDtypeStruct((N, D), jnp.float32),
    )(wv[0], wv[1], den2[0], den2[1], skip, Wo, Wg3, g1, b1, g2, b2, W1, W2)


def kernel(x, edge_index, Wq, Wk, Wv, Wo, Wskip, Wg, g1, b1, g2, b2, W1, W2):
    src = edge_index[0].astype(jnp.int32)
    dst = edge_index[1].astype(jnp.int32)
    q, k, v, skip = _proj(x, Wq, Wk, Wv, Wskip)
    wv, den = _edge(q, k, v, src, dst)
    den2 = den.reshape(NC, NP, 16)
    Wg3 = Wg[:, 0].reshape(3, D)
    return _epilogue(wv, den2, skip, Wo, Wg3,
                     g1.reshape(1, D), b1.reshape(1, D),
                     g2.reshape(1, D), b2.reshape(1, D), W1, W2)


# speedup vs baseline: 58.8424x; 1.0859x over previous
import functools

import jax
import jax.numpy as jnp
from jax import lax
from jax.experimental import pallas as pl
from jax.experimental.pallas import tpu as pltpu
from jax.experimental.pallas import tpu_sc as plsc

N = 10000
E = 320000
D = 128
H = 8
HD = 16
QSCALE = float(HD ** 0.5)

NC = 2
NS = 16
NW = NC * NS
EPW = E // NW
CH = 80
GPC = CH // 16
BCH = 400
CPB = BCH // CH
NBATCH = EPW // BCH
NP = 10240
RPT = NP // NS
DR = NP // 8
DPT = DR // NS
ZB = 40
NZB = RPT // ZB
DZB = DPT // ZB


def _proj_body(x_ref, wq_ref, wk_ref, wv_ref, ws_ref, q_ref, k_ref, v_ref,
               s_ref):
    xb = x_ref[...]
    q_ref[...] = QSCALE * jnp.dot(
        xb, wq_ref[...], preferred_element_type=jnp.float32)
    k_ref[...] = jnp.dot(xb, wk_ref[...], preferred_element_type=jnp.float32)
    v_ref[...] = jnp.dot(xb, wv_ref[...], preferred_element_type=jnp.float32)
    s_ref[...] = jnp.dot(xb, ws_ref[...], preferred_element_type=jnp.float32)


def _proj(x, Wq, Wk, Wv, Wskip, bm=2000):
    grid = (N // bm,)
    blk = pl.BlockSpec((bm, D), lambda i: (i, 0))
    wblk = pl.BlockSpec((D, D), lambda i: (0, 0))
    out = jax.ShapeDtypeStruct((N, D), jnp.float32)
    return pl.pallas_call(
        _proj_body,
        grid=grid,
        in_specs=[blk, wblk, wblk, wblk, wblk],
        out_specs=[blk, blk, blk, blk],
        out_shape=[out, out, out, out],
    )(x, Wq, Wk, Wv, Wskip)


_GDN = lax.GatherDimensionNumbers(
    offset_dims=(), collapsed_slice_dims=(0,), start_index_map=(0,))


def _shuffle(x, idx):
    return lax.gather(x, idx.reshape(16, 1), _GDN, (1,),
                      mode=lax.GatherScatterMode.PROMISE_IN_BOUNDS)


def _bcast_lane(x, j):
    return _shuffle(x, jnp.full((16,), j, dtype=jnp.int32))


POS = (0, 8, 4, 12, 2, 10, 6, 14)


def _edge_body(q_hbm, k_hbm, v_hbm, src_hbm, dst_hbm, wv_out, den_out,
               acc_wv, acc_den, src_b, dst_b, src_v, dst_v, ddiv_v, band_v,
               qr, kr, vr, bounce, sem):
    c = lax.axis_index("c")
    s = lax.axis_index("s")
    wid = s * NC + c
    lane = lax.iota(jnp.int32, 16)
    zvec = jnp.zeros((16,), jnp.float32)
    onev = jnp.ones((16,), jnp.float32)
    idxden = ((lane & 1) << 3) | ((lane & 2) << 1) | ((lane & 4) >> 1)

    def zrow(r, _):
        for cb in range(D // 16):
            bounce[r, pl.ds(16 * cb, 16)] = zvec
        return 0

    lax.fori_loop(0, ZB, zrow, 0)
    r0 = s * RPT
    for blk in range(NZB):
        pltpu.sync_copy(bounce, acc_wv.at[pl.ds(r0 + blk * ZB, ZB)])
    for blk in range(DZB):
        pltpu.sync_copy(bounce, acc_den.at[pl.ds(s * DPT + blk * ZB, ZB)])
    plsc.subcore_barrier()

    e_base = wid * EPW

    def batch(b, _):
        off = pl.multiple_of(e_base + b * BCH, 8)
        pltpu.sync_copy(src_hbm.at[pl.ds(off, BCH)], src_b)
        pltpu.sync_copy(dst_hbm.at[pl.ds(off, BCH)], dst_b)

        def chunk(i, _):
            c0 = i * CH
            for w in range(GPC):
                src_v[pl.ds(16 * w, 16)] = src_b[pl.ds(c0 + 16 * w, 16)]
                dst_v[pl.ds(16 * w, 16)] = dst_b[pl.ds(c0 + 16 * w, 16)]
            cp_q = pltpu.async_copy(q_hbm.at[src_v], qr, sem)
            cp_k = pltpu.async_copy(k_hbm.at[dst_v], kr, sem)
            cp_v = pltpu.async_copy(v_hbm.at[src_v], vr, sem)
            cp_q.wait()
            cp_k.wait()
            cp_v.wait()
            _chunk_compute(dst_v, ddiv_v, band_v, qr, kr, vr, lane, zvec,
                           onev, idxden)
            pltpu.sync_copy(vr, acc_wv.at[dst_v], add=True)
            pltpu.sync_copy(qr, acc_den.at[ddiv_v], add=True)
            return 0

        lax.fori_loop(0, CPB, chunk, 0)
        return 0

    lax.fori_loop(0, NBATCH, batch, 0)
    plsc.subcore_barrier()

    for blk in range(NZB):
        rr = r0 + blk * ZB
        pltpu.sync_copy(acc_wv.at[pl.ds(rr, ZB)], bounce)
        pltpu.sync_copy(bounce, wv_out.at[c, pl.ds(rr, ZB)])
    for blk in range(DZB):
        dd = s * DPT + blk * ZB
        pltpu.sync_copy(acc_den.at[pl.ds(dd, ZB)], bounce)
        pltpu.sync_copy(bounce, den_out.at[c, pl.ds(dd, ZB)])


def _chunk_compute(dst_v, ddiv_v, band_v, qr, kr, vr, lane, zvec, onev,
                   idxden):
    m8 = lane < 8
    m4 = (lane & 7) < 4
    m2 = (lane & 3) < 2

    def group(g, _):
        dstg = dst_v[pl.ds(g * 16, 16)]
        ddiv_v[pl.ds(g * 16, 16)] = lax.shift_right_logical(dstg, 3)
        band_v[pl.ds(0, 16)] = dstg & 7
        bandall = band_v[pl.ds(0, 16)]
        eqf = [jnp.where(bandall == cb, onev, zvec) for cb in range(8)]
        for j in range(16):
            e = g * 16 + j
            aa = [qr[e, pl.ds(16 * h, 16)] * kr[e, pl.ds(16 * h, 16)]
                  for h in range(H)]
            aa = [a + _shuffle(a, lane ^ 8) for a in aa]
            bb = [jnp.where(m8, aa[2 * i], _shuffle(aa[2 * i + 1], lane ^ 8))
                  for i in range(4)]
            bb = [b + _shuffle(b, lane ^ 4) for b in bb]
            cc = [jnp.where(m4, bb[2 * i], _shuffle(bb[2 * i + 1], lane ^ 4))
                  for i in range(2)]
            cc = [c2 + _shuffle(c2, lane ^ 2) for c2 in cc]
            f = jnp.where(m2, cc[0], _shuffle(cc[1], lane ^ 2))
            f = f + _shuffle(f, lane ^ 1)
            exm = jnp.exp(jnp.clip(f, -20.0, 20.0))
            for h in range(H):
                bch = _bcast_lane(exm, POS[h])
                vr[e, pl.ds(16 * h, 16)] = bch * vr[e, pl.ds(16 * h, 16)]
            den_acc = _shuffle(exm, idxden)
            for cb in range(8):
                mb = _bcast_lane(eqf[cb], j)
                qr[e, pl.ds(16 * cb, 16)] = mb * den_acc
        return 0

    lax.fori_loop(0, GPC, group, 0)


_edge = pl.kernel(
    _edge_body,
    out_type=[
        jax.ShapeDtypeStruct((NC, NP, D), jnp.float32),
        jax.ShapeDtypeStruct((NC, DR, D), jnp.float32),
    ],
    mesh=plsc.VectorSubcoreMesh(core_axis_name="c", subcore_axis_name="s"),
    scratch_types=[
        pltpu.VMEM_SHARED((NP, D), jnp.float32),
        pltpu.VMEM_SHARED((DR, D), jnp.float32),
        pltpu.VMEM((BCH,), jnp.int32),
        pltpu.VMEM((BCH,), jnp.int32),
        pltpu.VMEM((CH,), jnp.int32),
        pltpu.VMEM((CH,), jnp.int32),
        pltpu.VMEM((CH,), jnp.int32),
        pltpu.VMEM((16,), jnp.int32),
        pltpu.VMEM((CH, D), jnp.float32),
        pltpu.VMEM((CH, D), jnp.float32),
        pltpu.VMEM((CH, D), jnp.float32),
        pltpu.VMEM((ZB, D), jnp.float32),
        pltpu.SemaphoreType.DMA,
    ],
)


def _epi_body(wv0_ref, wv1_ref, d0_ref, d1_ref, skip_ref, wo_ref, wg_ref,
              g1_ref, b1_ref, g2_ref, b2_ref, w1_ref, w2_ref, out_ref):
    den16 = d0_ref[...] + d1_ref[...]
    jrow = lax.broadcasted_iota(jnp.int32, (16, D), 0)
    lcol = lax.broadcasted_iota(jnp.int32, (16, D), 1) // HD
    emat = (jrow == lcol).astype(jnp.float32)
    dexp = jnp.dot(den16, emat, preferred_element_type=jnp.float32)
    dexp = jnp.where(dexp > 0.0, dexp, 1.0)
    agg = (wv0_ref[...] + wv1_ref[...]) / dexp

    rst = jnp.dot(agg, wo_ref[...], preferred_element_type=jnp.float32)
    skip = skip_ref[...]
    wg = wg_ref[...]
    ga = wg[0] + wg[2]
    gb = wg[1] - wg[2]
    glog = (jnp.sum(rst * ga, axis=-1, keepdims=True)
            + jnp.sum(skip * gb, axis=-1, keepdims=True))
    gate = 1.0 / (1.0 + jnp.exp(-glog))
    hmid = rst * gate + skip * (1.0 - gate)

    mu = jnp.mean(hmid, axis=-1, keepdims=True)
    var = jnp.mean((hmid - mu) ** 2, axis=-1, keepdims=True)
    hn = (hmid - mu) * lax.rsqrt(var + 1e-5) * g1_ref[...] + b1_ref[...]

    mu2 = jnp.mean(hn, axis=-1, keepdims=True)
    var2 = jnp.mean((hn - mu2) ** 2, axis=-1, keepdims=True)
    h2 = (hn - mu2) * lax.rsqrt(var2 + 1e-5) * g2_ref[...] + b2_ref[...]

    ffn = jnp.dot(
        jnp.maximum(
            jnp.dot(h2, w1_ref[...], preferred_element_type=jnp.float32), 0.0),
        w2_ref[...], preferred_element_type=jnp.float32)
    out_ref[...] = hn + ffn


def _epilogue(wv, den2, skip, Wo, Wg3, g1, b1, g2, b2, W1, W2, bm=2000):
    grid = (N // bm,)
    blk = pl.BlockSpec((bm, D), lambda i: (i, 0))
    dblk = pl.BlockSpec((bm, 16), lambda i: (i, 0))
    wblk = pl.BlockSpec((D, D), lambda i: (0, 0))
    gblk = pl.BlockSpec((3, D), lambda i: (0, 0))
    vblk = pl.BlockSpec((1, D), lambda i: (0, 0))
    return pl.pallas_call(
        _epi_body,
        grid=grid,
        in_specs=[blk, blk, dblk, dblk, blk, wblk, gblk,
                  vblk, vblk, vblk, vblk, wblk, wblk],
        out_specs=blk,
        out_shape=jax.ShapeDtypeStruct((N, D), jnp.float32),
    )(wv[0], wv[1], den2[0], den2[1], skip, Wo, Wg3, g1, b1, g2, b2, W1, W2)


def kernel(x, edge_index, Wq, Wk, Wv, Wo, Wskip, Wg, g1, b1, g2, b2, W1, W2):
    src = edge_index[0].astype(jnp.int32)
    dst = edge_index[1].astype(jnp.int32)
    q, k, v, skip = _proj(x, Wq, Wk, Wv, Wskip)
    wv, den = _edge(q, k, v, src, dst)
    den2 = den.reshape(NC, NP, 16)
    Wg3 = Wg[:, 0].reshape(3, D)
    return _epilogue(wv, den2, skip, Wo, Wg3,
                     g1.reshape(1, D), b1.reshape(1, D),
                     g2.reshape(1, D), b2.reshape(1, D), W1, W2)


# speedup vs baseline: 59.4067x; 1.0096x over previous
import functools

import jax
import jax.numpy as jnp
from jax import lax
from jax.experimental import pallas as pl
from jax.experimental.pallas import tpu as pltpu
from jax.experimental.pallas import tpu_sc as plsc

N = 10000
E = 320000
D = 128
H = 8
HD = 16
QSCALE = float(HD ** 0.5)

NC = 2
NS = 16
NW = NC * NS
EPW = E // NW
CH = 80
GPC = CH // 16
BCH = 400
CPB = BCH // CH
NBATCH = EPW // BCH
NP = 10240
RPT = NP // NS
DR = NP // 8
DPT = DR // NS
ZB = 40
NZB = RPT // ZB
DZB = DPT // ZB


def _proj_body(x_ref, wq_ref, wk_ref, wv_ref, ws_ref, q_ref, k_ref, v_ref,
               s_ref):
    xb = x_ref[...]
    q_ref[...] = QSCALE * jnp.dot(
        xb, wq_ref[...], preferred_element_type=jnp.float32)
    k_ref[...] = jnp.dot(xb, wk_ref[...], preferred_element_type=jnp.float32)
    v_ref[...] = jnp.dot(xb, wv_ref[...], preferred_element_type=jnp.float32)
    s_ref[...] = jnp.dot(xb, ws_ref[...], preferred_element_type=jnp.float32)


def _proj(x, Wq, Wk, Wv, Wskip, bm=2000):
    grid = (N // bm,)
    blk = pl.BlockSpec((bm, D), lambda i: (i, 0))
    wblk = pl.BlockSpec((D, D), lambda i: (0, 0))
    out = jax.ShapeDtypeStruct((N, D), jnp.float32)
    return pl.pallas_call(
        _proj_body,
        grid=grid,
        in_specs=[blk, wblk, wblk, wblk, wblk],
        out_specs=[blk, blk, blk, blk],
        out_shape=[out, out, out, out],
    )(x, Wq, Wk, Wv, Wskip)


_GDN = lax.GatherDimensionNumbers(
    offset_dims=(), collapsed_slice_dims=(0,), start_index_map=(0,))


def _shuffle(x, idx):
    return lax.gather(x, idx.reshape(16, 1), _GDN, (1,),
                      mode=lax.GatherScatterMode.PROMISE_IN_BOUNDS)


def _bcast_lane(x, j):
    return _shuffle(x, jnp.full((16,), j, dtype=jnp.int32))


POS = (0, 8, 4, 12, 2, 10, 6, 14)


def _edge_body(q_hbm, k_hbm, v_hbm, src_hbm, dst_hbm, wv_out, den_out,
               acc_wv, acc_den, src_b, dst_b, src_v, dst_v, ddiv_v, band_v,
               qr, kr, vr, bounce, sem):
    c = lax.axis_index("c")
    s = lax.axis_index("s")
    wid = s * NC + c
    lane = lax.iota(jnp.int32, 16)
    zvec = jnp.zeros((16,), jnp.float32)
    onev = jnp.ones((16,), jnp.float32)
    idxden = ((lane & 1) << 3) | ((lane & 2) << 1) | ((lane & 4) >> 1)

    def zrow(r, _):
        for cb in range(D // 16):
            bounce[r, pl.ds(16 * cb, 16)] = zvec
        return 0

    lax.fori_loop(0, ZB, zrow, 0)
    r0 = s * RPT
    for blk in range(NZB):
        pltpu.sync_copy(bounce, acc_wv.at[pl.ds(r0 + blk * ZB, ZB)])
    for blk in range(DZB):
        pltpu.sync_copy(bounce, acc_den.at[pl.ds(s * DPT + blk * ZB, ZB)])
    plsc.subcore_barrier()

    e_base = wid * EPW

    def batch(b, _):
        off = pl.multiple_of(e_base + b * BCH, 8)
        pltpu.sync_copy(src_hbm.at[pl.ds(off, BCH)], src_b)
        pltpu.sync_copy(dst_hbm.at[pl.ds(off, BCH)], dst_b)

        def chunk(i, _):
            c0 = i * CH
            for w in range(GPC):
                src_v[pl.ds(16 * w, 16)] = src_b[pl.ds(c0 + 16 * w, 16)]
                dst_v[pl.ds(16 * w, 16)] = dst_b[pl.ds(c0 + 16 * w, 16)]
            cp_q = pltpu.async_copy(q_hbm.at[src_v], qr, sem)
            cp_k = pltpu.async_copy(k_hbm.at[dst_v], kr, sem)
            cp_v = pltpu.async_copy(v_hbm.at[src_v], vr, sem)
            cp_q.wait()
            cp_k.wait()
            cp_v.wait()
            _chunk_compute(dst_v, ddiv_v, band_v, qr, kr, vr, lane, zvec,
                           onev, idxden)
            cp_wv = pltpu.async_copy(vr, acc_wv.at[dst_v], sem, add=True)
            cp_dn = pltpu.async_copy(qr, acc_den.at[ddiv_v], sem, add=True)
            cp_wv.wait()
            cp_dn.wait()
            return 0

        lax.fori_loop(0, CPB, chunk, 0)
        return 0

    lax.fori_loop(0, NBATCH, batch, 0)
    plsc.subcore_barrier()

    for blk in range(NZB):
        rr = r0 + blk * ZB
        pltpu.sync_copy(acc_wv.at[pl.ds(rr, ZB)], bounce)
        pltpu.sync_copy(bounce, wv_out.at[c, pl.ds(rr, ZB)])
    for blk in range(DZB):
        dd = s * DPT + blk * ZB
        pltpu.sync_copy(acc_den.at[pl.ds(dd, ZB)], bounce)
        pltpu.sync_copy(bounce, den_out.at[c, pl.ds(dd, ZB)])


def _chunk_compute(dst_v, ddiv_v, band_v, qr, kr, vr, lane, zvec, onev,
                   idxden):
    m8 = lane < 8
    m4 = (lane & 7) < 4
    m2 = (lane & 3) < 2

    def group(g, _):
        dstg = dst_v[pl.ds(g * 16, 16)]
        ddiv_v[pl.ds(g * 16, 16)] = lax.shift_right_logical(dstg, 3)
        band_v[pl.ds(0, 16)] = dstg & 7
        bandall = band_v[pl.ds(0, 16)]
        eqf = [jnp.where(bandall == cb, onev, zvec) for cb in range(8)]
        for j in range(16):
            e = g * 16 + j
            aa = [qr[e, pl.ds(16 * h, 16)] * kr[e, pl.ds(16 * h, 16)]
                  for h in range(H)]
            aa = [a + _shuffle(a, lane ^ 8) for a in aa]
            bb = [jnp.where(m8, aa[2 * i], _shuffle(aa[2 * i + 1], lane ^ 8))
                  for i in range(4)]
            bb = [b + _shuffle(b, lane ^ 4) for b in bb]
            cc = [jnp.where(m4, bb[2 * i], _shuffle(bb[2 * i + 1], lane ^ 4))
                  for i in range(2)]
            cc = [c2 + _shuffle(c2, lane ^ 2) for c2 in cc]
            f = jnp.where(m2, cc[0], _shuffle(cc[1], lane ^ 2))
            f = f + _shuffle(f, lane ^ 1)
            exm = jnp.exp(jnp.clip(f, -20.0, 20.0))
            for h in range(H):
                bch = _bcast_lane(exm, POS[h])
                vr[e, pl.ds(16 * h, 16)] = bch * vr[e, pl.ds(16 * h, 16)]
            den_acc = _shuffle(exm, idxden)
            for cb in range(8):
                mb = _bcast_lane(eqf[cb], j)
                qr[e, pl.ds(16 * cb, 16)] = mb * den_acc
        return 0

    lax.fori_loop(0, GPC, group, 0)


_edge = pl.kernel(
    _edge_body,
    out_type=[
        jax.ShapeDtypeStruct((NC, NP, D), jnp.float32),
        jax.ShapeDtypeStruct((NC, DR, D), jnp.float32),
    ],
    mesh=plsc.VectorSubcoreMesh(core_axis_name="c", subcore_axis_name="s"),
    scratch_types=[
        pltpu.VMEM_SHARED((NP, D), jnp.float32),
        pltpu.VMEM_SHARED((DR, D), jnp.float32),
        pltpu.VMEM((BCH,), jnp.int32),
        pltpu.VMEM((BCH,), jnp.int32),
        pltpu.VMEM((CH,), jnp.int32),
        pltpu.VMEM((CH,), jnp.int32),
        pltpu.VMEM((CH,), jnp.int32),
        pltpu.VMEM((16,), jnp.int32),
        pltpu.VMEM((CH, D), jnp.float32),
        pltpu.VMEM((CH, D), jnp.float32),
        pltpu.VMEM((CH, D), jnp.float32),
        pltpu.VMEM((ZB, D), jnp.float32),
        pltpu.SemaphoreType.DMA,
    ],
)


def _epi_body(wv0_ref, wv1_ref, d0_ref, d1_ref, skip_ref, wo_ref, wg_ref,
              g1_ref, b1_ref, g2_ref, b2_ref, w1_ref, w2_ref, out_ref):
    den16 = d0_ref[...] + d1_ref[...]
    jrow = lax.broadcasted_iota(jnp.int32, (16, D), 0)
    lcol = lax.broadcasted_iota(jnp.int32, (16, D), 1) // HD
    emat = (jrow == lcol).astype(jnp.float32)
    dexp = jnp.dot(den16, emat, preferred_element_type=jnp.float32)
    dexp = jnp.where(dexp > 0.0, dexp, 1.0)
    agg = (wv0_ref[...] + wv1_ref[...]) / dexp

    rst = jnp.dot(agg, wo_ref[...], preferred_element_type=jnp.float32)
    skip = skip_ref[...]
    wg = wg_ref[...]
    ga = wg[0] + wg[2]
    gb = wg[1] - wg[2]
    glog = (jnp.sum(rst * ga, axis=-1, keepdims=True)
            + jnp.sum(skip * gb, axis=-1, keepdims=True))
    gate = 1.0 / (1.0 + jnp.exp(-glog))
    hmid = rst * gate + skip * (1.0 - gate)

    mu = jnp.mean(hmid, axis=-1, keepdims=True)
    var = jnp.mean((hmid - mu) ** 2, axis=-1, keepdims=True)
    hn = (hmid - mu) * lax.rsqrt(var + 1e-5) * g1_ref[...] + b1_ref[...]

    mu2 = jnp.mean(hn, axis=-1, keepdims=True)
    var2 = jnp.mean((hn - mu2) ** 2, axis=-1, keepdims=True)
    h2 = (hn - mu2) * lax.rsqrt(var2 + 1e-5) * g2_ref[...] + b2_ref[...]

    ffn = jnp.dot(
        jnp.maximum(
            jnp.dot(h2, w1_ref[...], preferred_element_type=jnp.float32), 0.0),
        w2_ref[...], preferred_element_type=jnp.float32)
    out_ref[...] = hn + ffn


def _epilogue(wv, den2, skip, Wo, Wg3, g1, b1, g2, b2, W1, W2, bm=2000):
    grid = (N // bm,)
    blk = pl.BlockSpec((bm, D), lambda i: (i, 0))
    dblk = pl.BlockSpec((bm, 16), lambda i: (i, 0))
    wblk = pl.BlockSpec((D, D), lambda i: (0, 0))
    gblk = pl.BlockSpec((3, D), lambda i: (0, 0))
    vblk = pl.BlockSpec((1, D), lambda i: (0, 0))
    return pl.pallas_call(
        _epi_body,
        grid=grid,
        in_specs=[blk, blk, dblk, dblk, blk, wblk, gblk,
                  vblk, vblk, vblk, vblk, wblk, wblk],
        out_specs=blk,
        out_shape=jax.ShapeDtypeStruct((N, D), jnp.float32),
    )(wv[0], wv[1], den2[0], den2[1], skip, Wo, Wg3, g1, b1, g2, b2, W1, W2)


def kernel(x, edge_index, Wq, Wk, Wv, Wo, Wskip, Wg, g1, b1, g2, b2, W1, W2):
    src = edge_index[0].astype(jnp.int32)
    dst = edge_index[1].astype(jnp.int32)
    q, k, v, skip = _proj(x, Wq, Wk, Wv, Wskip)
    wv, den = _edge(q, k, v, src, dst)
    den2 = den.reshape(NC, NP, 16)
    Wg3 = Wg[:, 0].reshape(3, D)
    return _epilogue(wv, den2, skip, Wo, Wg3,
                     g1.reshape(1, D), b1.reshape(1, D),
                     g2.reshape(1, D), b2.reshape(1, D), W1, W2)
